# baseline (speedup 1.0000x reference)
import functools

import jax
import jax.numpy as jnp
from jax import lax
from jax.experimental import pallas as pl
from jax.experimental.pallas import tpu as pltpu
from jax.experimental.pallas import tpu_sc as plsc

NC = 2
NS = 16
L = 16

N = 10000
E = 160000
NPAD = 10240
HALF = NPAD // NC

E_PER_TILE = E // NS
IDX_ROWS = (E_PER_TILE + 127) // 128 + 1


def _sc_mesh():
  return plsc.VectorSubcoreMesh(core_axis_name="c", subcore_axis_name="s")


_E_PER_W = E // (NC * NS)
_DEG_FULL = _E_PER_W // L
_DEG_TAIL = _E_PER_W - _DEG_FULL * L


def _deg_body(dst_hbm, degp_hbm, dst_v, cnt):
  c = lax.axis_index("c")
  s = lax.axis_index("s")
  wid = c * NS + s
  pltpu.sync_copy(dst_hbm.at[pl.ds(wid * _E_PER_W, _E_PER_W)],
                  dst_v.at[pl.ds(0, _E_PER_W)])

  def zero(i, carry):
    cnt[pl.ds(i * L, L)] = jnp.zeros((L,), jnp.float32)
    return carry
  lax.fori_loop(0, 640, zero, 0)

  ones = jnp.ones((L,), jnp.float32)

  def count(i, carry):
    d = dst_v[pl.ds(i * L, L)]
    plsc.addupdate_scatter(cnt, [d], ones)
    return carry
  lax.fori_loop(0, _DEG_FULL, count, 0)

  if _DEG_TAIL:
    d = dst_v[pl.ds(_DEG_FULL * L, L)]
    msk = lax.iota(jnp.int32, L) < _DEG_TAIL
    d = jnp.where(msk, d, 0)
    plsc.addupdate_scatter(cnt, [d], ones, mask=msk)

  pltpu.sync_copy(cnt, degp_hbm.at[wid])


_SC_PARAMS = pltpu.CompilerParams(needs_layout_passes=False)

_deg_kernel = functools.partial(
    pl.kernel,
    out_type=jax.ShapeDtypeStruct((NC * NS, 640 * 16), jnp.float32),
    mesh=_sc_mesh(),
    compiler_params=_SC_PARAMS,
    scratch_types=[
        pltpu.VMEM((_E_PER_W + L,), jnp.int32),
        pltpu.VMEM((640 * 16,), jnp.float32),
    ],
)(_deg_body)


def _make_agg(D, WR):
  NWIN = HALF // WR
  SHIFT = WR.bit_length() - 1
  scan_iters = E_PER_TILE // L
  NB = -(-(NWIN + 1) // L)
  SENT = 63
  ZR = 8
  CG = 32

  def body(src_hbm, dst_hbm, h_hbm, out_hbm,
           src_v, dst_v, srcS, dofS, hist, start, cur, tmp16, tmpw, idx_gc,
           rows, win, zbuf, sem):
    c = lax.axis_index("c")
    s = lax.axis_index("s")
    iota = lax.iota(jnp.int32, L)
    ones_i = jnp.ones((L,), jnp.int32)
    gbase = c * HALF

    def sca(vec, lane):
      return jnp.sum(jnp.where(iota == lane, vec, jnp.int32(0)))

    pltpu.sync_copy(src_hbm.at[pl.ds(s * E_PER_TILE, E_PER_TILE)], src_v)
    pltpu.sync_copy(dst_hbm.at[pl.ds(s * E_PER_TILE, E_PER_TILE)], dst_v)

    def zz(i, carry):
      r = i // (D // L)
      col = i % (D // L)
      zbuf[r, pl.ds(col * L, L)] = jnp.zeros((L,), jnp.float32)
      return carry
    lax.fori_loop(0, ZR * (D // L), zz, 0)
    for t in range(4):
      hist[pl.ds(t * L, L)] = jnp.zeros((L,), jnp.int32)

    rt = HALF // NS

    def zo(kk, carry):
      pltpu.sync_copy(zbuf, out_hbm.at[pl.ds(gbase + s * rt + kk * ZR, ZR)])
      return carry
    lax.fori_loop(0, rt // ZR, zo, 0)

    def h1(i, carry):
      dv = dst_v[pl.ds(i * L, L)]
      dof = dv - gbase
      inhalf = (dof >= 0) & (dof < HALF)
      w = jnp.where(inhalf, dof >> SHIFT, SENT)
      plsc.addupdate_scatter(hist, [w], ones_i)
      return carry
    lax.fori_loop(0, scan_iters, h1, 0)

    carry = jnp.int32(0)
    for t in range(NB):
      hv = hist[pl.ds(t * L, L)]
      inc = jnp.cumsum(hv)
      start[pl.ds(t * L, L)] = carry + inc - hv
      cur[pl.ds(t * L, L)] = carry + inc - hv
      carry = carry + jnp.sum(hv)

    def h2(i, carry):
      sv = src_v[pl.ds(i * L, L)]
      dv = dst_v[pl.ds(i * L, L)]
      dof = dv - gbase
      inhalf = (dof >= 0) & (dof < HALF)
      w = jnp.where(inhalf, dof >> SHIFT, SENT)
      cnt = plsc.load_gather(cur, [w])
      wk, lidx = plsc.sort_key_val(w, iota)
      tmpw[...] = wk
      prev = plsc.load_gather(tmpw, [jnp.maximum(iota - 1, 0)])
      segstart = plsc.cummax(jnp.where(wk != prev, iota, 0))
      plsc.store_scatter(tmp16, [lidx], iota - segstart)
      pos = cnt + tmp16[...]
      plsc.store_scatter(srcS, [pos], sv, mask=inhalf)
      plsc.store_scatter(dofS, [pos], dof, mask=inhalf)
      plsc.addupdate_scatter(cur, [w], ones_i)
      return carry
    lax.fori_loop(0, scan_iters, h2, 0)

    ktot = carry
    for t in range(CG // L):
      srcS[pl.ds(ktot + t * L, L)] = jnp.zeros((L,), jnp.int32)

    plsc.subcore_barrier()

    def rot(k, rcarry):
      w = lax.rem(s + k, NWIN)
      sv = start[pl.ds(w, L)]
      e0 = sca(sv, 0)
      ne = sca(sv, 1) - e0

      @pl.when(ne > 0)
      def _visit():
        wbase = gbase + w * WR
        pltpu.sync_copy(out_hbm.at[pl.ds(wbase, WR)], win)

        nch = (ne + CG - 1) // CG

        def chunk(j, carry):
          for t in range(CG // L):
            idx_gc[pl.ds(t * L, L)] = srcS[pl.ds(e0 + j * CG + t * L, L)]
          pltpu.async_copy(h_hbm.at[idx_gc], rows, sem).wait()
          nedge = jnp.minimum(ne - j * CG, CG)

          def edge(i, carry2):
            dvv = dofS[pl.ds(e0 + j * CG + i, L)]
            row = sca(dvv, 0) - w * WR
            for g in range(D // L):
              plsc.addupdate(win.at[row, pl.ds(g * L, L)],
                             rows[i, pl.ds(g * L, L)])
            return carry2
          lax.fori_loop(0, nedge, edge, 0)
          return carry
        lax.fori_loop(0, nch, chunk, 0)

        pltpu.sync_copy(win, out_hbm.at[pl.ds(wbase, WR)])

      plsc.subcore_barrier()
      return rcarry
    lax.fori_loop(0, NWIN, rot, 0)

  return pl.kernel(
      body,
      out_type=jax.ShapeDtypeStruct((NPAD, D), jnp.float32),
      mesh=_sc_mesh(),
      compiler_params=_SC_PARAMS,
      scratch_types=[
          pltpu.VMEM((E_PER_TILE,), jnp.int32),
          pltpu.VMEM((E_PER_TILE,), jnp.int32),
          pltpu.VMEM((E_PER_TILE + 64,), jnp.int32),
          pltpu.VMEM((E_PER_TILE + 64,), jnp.int32),
          pltpu.VMEM((64,), jnp.int32),
          pltpu.VMEM((64,), jnp.int32),
          pltpu.VMEM((64,), jnp.int32),
          pltpu.VMEM((L,), jnp.int32),
          pltpu.VMEM((L,), jnp.int32),
          pltpu.VMEM((CG,), jnp.int32),
          pltpu.VMEM((CG, D), jnp.float32),
          pltpu.VMEM((WR, D), jnp.float32),
          pltpu.VMEM((ZR, D), jnp.float32),
          pltpu.SemaphoreType.DMA,
      ],
  )


MBLK = 400


def _dinv_from_partials(degp_blk):
  deg = jnp.sum(degp_blk, axis=1) + 1.0
  return lax.rsqrt(deg)


def _mm1_body(x_ref, w_ref, degp_ref, o_ref):
  dinv = _dinv_from_partials(degp_ref[...])
  h = jnp.dot(x_ref[...], w_ref[...], preferred_element_type=jnp.float32)
  o_ref[...] = h * dinv[:, None]


def _mm2_body(agg_ref, h_ref, degp_ref, b_ref, w_ref, o_ref):
  dinv = _dinv_from_partials(degp_ref[...])
  z = jnp.maximum(dinv[:, None] * (agg_ref[...] + h_ref[...]) + b_ref[...],
                  0.0)
  h2 = jnp.dot(z, w_ref[...], preferred_element_type=jnp.float32)
  o_ref[...] = h2 * dinv[:, None]


def _final_body(agg_ref, h_ref, degp_ref, b_ref, o_ref):
  dinv = _dinv_from_partials(degp_ref[...])
  o = dinv[:, None] * (agg_ref[...] + h_ref[...]) + b_ref[...]
  o = o - jnp.max(o, axis=1, keepdims=True)
  e = jnp.exp(o)
  o_ref[...] = e / jnp.sum(e, axis=1, keepdims=True)


def _mm1(x, w1, degp):
  d_in, d_hid = w1.shape
  return pl.pallas_call(
      _mm1_body,
      grid=(N // MBLK,),
      in_specs=[
          pl.BlockSpec((MBLK, d_in), lambda i: (i, 0)),
          pl.BlockSpec((d_in, d_hid), lambda i: (0, 0)),
          pl.BlockSpec((MBLK, NC * NS), lambda i: (i, 0)),
      ],
      out_specs=pl.BlockSpec((MBLK, d_hid), lambda i: (i, 0)),
      out_shape=jax.ShapeDtypeStruct((N, d_hid), jnp.float32),
  )(x, w1, degp)


def _mm2(agg1, h1p, degp, b1, w2):
  d_hid, d_out = w2.shape
  return pl.pallas_call(
      _mm2_body,
      grid=(N // MBLK,),
      in_specs=[
          pl.BlockSpec((MBLK, d_hid), lambda i: (i, 0)),
          pl.BlockSpec((MBLK, d_hid), lambda i: (i, 0)),
          pl.BlockSpec((MBLK, NC * NS), lambda i: (i, 0)),
          pl.BlockSpec((1, d_hid), lambda i: (0, 0)),
          pl.BlockSpec((d_hid, d_out), lambda i: (0, 0)),
      ],
      out_specs=pl.BlockSpec((MBLK, d_out), lambda i: (i, 0)),
      out_shape=jax.ShapeDtypeStruct((N, d_out), jnp.float32),
  )(agg1, h1p, degp, b1, w2)


def _final(agg2, h2p, degp, b2):
  d_out = h2p.shape[1]
  return pl.pallas_call(
      _final_body,
      grid=(N // MBLK,),
      in_specs=[
          pl.BlockSpec((MBLK, d_out), lambda i: (i, 0)),
          pl.BlockSpec((MBLK, d_out), lambda i: (i, 0)),
          pl.BlockSpec((MBLK, NC * NS), lambda i: (i, 0)),
          pl.BlockSpec((1, d_out), lambda i: (0, 0)),
      ],
      out_specs=pl.BlockSpec((MBLK, d_out), lambda i: (i, 0)),
      out_shape=jax.ShapeDtypeStruct((N, d_out), jnp.float32),
  )(agg2, h2p, degp, b2)


_agg_512 = _make_agg(512, 128)
_agg_256 = _make_agg(256, 256)


@jax.jit
def kernel(x, edge_index, W1, b1, W2, b2):
  src = edge_index[0]
  dst = edge_index[1]

  degp = _deg_kernel(dst).T[:N]

  h1p = _mm1(x, W1, degp)
  agg1 = _agg_512(src, dst, h1p)[:N]
  h2p = _mm2(agg1, h1p, degp, b1.reshape(1, -1), W2)
  agg2 = _agg_256(src, dst, h2p)[:N]
  return _final(agg2, h2p, degp, b2.reshape(1, -1))
